# baseline (speedup 1.0000x reference)
import dataclasses
import functools

import jax
import jax.numpy as jnp
from jax import lax
from jax.experimental import pallas as pl
from jax.experimental.pallas import tpu as pltpu
from jax.experimental.pallas import tpu_sc as plsc

N = 10000
E = 320000
D = 128

NC = 2
NS = 16
NW = NC * NS
EPW = E // NW
W = 128
NWIN = (EPW // W)
WT = EPW - NWIN * W
K = 13
NCH = NWIN // K
KP = 16
NBUF = 2
ZW = 80
L = 16


def _sc_agg(feat, src, dstp, dst, with_deg):
    mesh = plsc.VectorSubcoreMesh(core_axis_name="c", subcore_axis_name="s")
    out_type = [jax.ShapeDtypeStruct((NC, N, D), jnp.float32)]
    scratch = [
        pltpu.VMEM_SHARED((N, D), jnp.float32),
        pltpu.VMEM((K * W,), jnp.int32),
        pltpu.VMEM((KP, W), jnp.int32),
        pltpu.VMEM((WT,), jnp.int32),
    ] + [pltpu.VMEM((W, D), jnp.float32) for _ in range(NBUF)] + [
        pltpu.SemaphoreType.DMA,
        pltpu.SemaphoreType.DMA,
    ]
    if with_deg:
        out_type.append(jax.ShapeDtypeStruct((NW, 1, N), jnp.float32))
        scratch.append(pltpu.VMEM((N,), jnp.float32))

    def body(*refs):
        if with_deg:
            (feat_h, src_h, dstp_h, dst_h, parts_h, degp_h,
             acc_sh, src_v, dst_v, tail_v, *rest) = refs
            rows = rest[:NBUF]
            sem = rest[NBUF]
            sem_s = rest[NBUF + 1]
            hist_v = rest[NBUF + 2]
        else:
            (feat_h, src_h, dstp_h, dst_h, parts_h,
             acc_sh, src_v, dst_v, tail_v, *rest) = refs
            rows = rest[:NBUF]
            sem = rest[NBUF]
            sem_s = rest[NBUF + 1]
        c = lax.axis_index("c")
        s = lax.axis_index("s")
        wid = s * NC + c

        @pl.loop(0, W)
        def _(i):
            @pl.loop(0, D, step=L)
            def _(j):
                rows[0].at[i, pl.ds(j, L)][...] = jnp.zeros((L,), jnp.float32)

        if with_deg:
            @pl.loop(0, N, step=L)
            def _(i):
                hist_v.at[pl.ds(i, L)][...] = jnp.zeros((L,), jnp.float32)

        nfull = (N // ZW) // NS

        def for_my_chunks(fn):
            for k in range(nfull):
                fn((s + k * NS) * ZW)
            tail = s + nfull * NS

            @pl.when(tail < N // ZW)
            def _():
                fn(tail * ZW)

        zsrc = rows[0].at[pl.ds(0, ZW)]
        for_my_chunks(lambda lo: pltpu.async_copy(
            zsrc, acc_sh.at[pl.ds(lo, ZW)], sem))
        for_my_chunks(lambda lo: pltpu.make_async_copy(
            zsrc, acc_sh.at[pl.ds(lo, ZW)], sem).wait())

        plsc.subcore_barrier()

        ones = jnp.full((L,), 1.0, jnp.float32)

        def do_scatter(j, buf):
            if with_deg:
                for k in range(W // L):
                    v = dst_v[j, pl.ds(k * L, L)]
                    plsc.addupdate_scatter(hist_v, [v], ones)
            return pltpu.async_copy(buf, acc_sh.at[dst_v.at[j]], sem_s,
                                    add=True)

        def fire(j):
            return pltpu.async_copy(
                feat_h.at[src_v.at[pl.ds(j * W, W)]], rows[j % NBUF], sem)

        @pl.loop(0, NCH)
        def _(g):
            pltpu.sync_copy(src_h.at[pl.ds(wid * EPW + g * K * W, K * W)],
                            src_v)
            pltpu.sync_copy(dstp_h.at[wid].at[g], dst_v)
            gd = [fire(0), None]
            sd = None
            for j in range(K):
                b = j % NBUF
                if sd is not None:
                    sd.wait()
                if j + 1 < K:
                    gd[(j + 1) % NBUF] = fire(j + 1)
                gd[b].wait()
                sd = do_scatter(j, rows[b])
            sd.wait()

        tbase = wid * EPW + NWIN * W
        pltpu.sync_copy(dst_h.at[pl.ds(tbase, WT)], tail_v)
        pltpu.sync_copy(src_h.at[pl.ds(tbase, WT)], src_v.at[pl.ds(0, WT)])
        pltpu.async_copy(
            feat_h.at[src_v.at[pl.ds(0, WT)]], rows[0].at[pl.ds(0, WT)],
            sem).wait()
        if with_deg:
            plsc.addupdate_scatter(hist_v, [tail_v[...]], ones)
        pltpu.sync_copy(rows[0].at[pl.ds(0, WT)], acc_sh.at[tail_v], add=True)

        plsc.subcore_barrier()

        for_my_chunks(lambda lo: pltpu.async_copy(
            acc_sh.at[pl.ds(lo, ZW)], parts_h.at[c].at[pl.ds(lo, ZW)], sem))
        if with_deg:
            outd = pltpu.async_copy(hist_v, degp_h.at[wid].at[0], sem)
        for_my_chunks(lambda lo: pltpu.make_async_copy(
            acc_sh.at[pl.ds(lo, ZW)], parts_h.at[c].at[pl.ds(lo, ZW)],
            sem).wait())
        if with_deg:
            outd.wait()

    cp = pltpu.CompilerParams()
    if "needs_layout_passes" in pltpu.CompilerParams.__dataclass_fields__:
        cp = dataclasses.replace(cp, needs_layout_passes=False)
    fn = pl.kernel(body, out_type=tuple(out_type), mesh=mesh,
                   scratch_types=scratch, compiler_params=cp)
    return fn(feat, src, dstp, dst)


_BN = 1000


def _tc0_body(x_ref, p_ref, d_ref, wa_ref, wb_ref, o_ref, r_out_ref, recip_s):
    i = pl.program_id(0)

    @pl.when(i == 0)
    def _():
        d = d_ref[...][:, 0, :]
        ones = jnp.ones((NW, 1), jnp.float32)
        deg = lax.dot_general(d, ones, (((0,), (0,)), ((), ())),
                              preferred_element_type=jnp.float32)
        recip_s[...] = jnp.broadcast_to(1.0 / jnp.maximum(deg, 1.0), (N, D))

    r = recip_s[pl.ds(i * _BN, _BN), :]
    r_out_ref[...] = r
    agg = (p_ref[0] + p_ref[1]) * r
    y = (jnp.dot(x_ref[...], wa_ref[...], preferred_element_type=jnp.float32)
         + jnp.dot(agg, wb_ref[...], preferred_element_type=jnp.float32))
    o_ref[...] = jnp.maximum(y, 0.0)


def _tc_layer0(x, parts, degp, wa_t, wb_t):
    return pl.pallas_call(
        _tc0_body,
        grid=(N // _BN,),
        in_specs=[
            pl.BlockSpec((_BN, D), lambda i: (i, 0)),
            pl.BlockSpec((NC, _BN, D), lambda i: (0, i, 0)),
            pl.BlockSpec((NW, 1, N), lambda i: (0, 0, 0)),
            pl.BlockSpec((D, D), lambda i: (0, 0)),
            pl.BlockSpec((D, D), lambda i: (0, 0)),
        ],
        out_specs=[
            pl.BlockSpec((_BN, D), lambda i: (i, 0)),
            pl.BlockSpec((_BN, D), lambda i: (i, 0)),
        ],
        out_shape=[
            jax.ShapeDtypeStruct((N, D), jnp.float32),
            jax.ShapeDtypeStruct((N, D), jnp.float32),
        ],
        scratch_shapes=[pltpu.VMEM((N, D), jnp.float32)],
    )(x, parts, degp, wa_t, wb_t)


def _tc_body(relu, x_ref, p_ref, r_ref, wa_ref, wb_ref, o_ref):
    agg = (p_ref[0] + p_ref[1]) * r_ref[...]
    y = (jnp.dot(x_ref[...], wa_ref[...], preferred_element_type=jnp.float32)
         + jnp.dot(agg, wb_ref[...], preferred_element_type=jnp.float32))
    if relu:
        y = jnp.maximum(y, 0.0)
    o_ref[...] = y


def _tc_layer(x, parts, recipb, wa_t, wb_t, relu):
    grid = (N // _BN,)
    return pl.pallas_call(
        functools.partial(_tc_body, relu),
        grid=grid,
        in_specs=[
            pl.BlockSpec((_BN, D), lambda i: (i, 0)),
            pl.BlockSpec((NC, _BN, D), lambda i: (0, i, 0)),
            pl.BlockSpec((_BN, D), lambda i: (i, 0)),
            pl.BlockSpec((D, D), lambda i: (0, 0)),
            pl.BlockSpec((D, D), lambda i: (0, 0)),
        ],
        out_specs=pl.BlockSpec((_BN, D), lambda i: (i, 0)),
        out_shape=jax.ShapeDtypeStruct((N, D), jnp.float32),
    )(x, parts, recipb, wa_t, wb_t)


def kernel(feat, edge_index, W0, W1):
    src = edge_index[0]
    dst = edge_index[1]
    dmain = dst.reshape(NW, EPW)[:, :NWIN * W].reshape(NW, NCH, K, W)
    dstp = jnp.pad(dmain, ((0, 0), (0, 0), (0, KP - K), (0, 0)))
    w0a_t = W0[:, :D].T
    w0b_t = W0[:, D:].T
    w1a_t = W1[:, :D].T
    w1b_t = W1[:, D:].T

    parts0, degp = _sc_agg(feat, src, dstp, dst, with_deg=True)
    h, recipb = _tc_layer0(feat, parts0, degp, w0a_t, w0b_t)
    parts1, = _sc_agg(h, src, dstp, dst, with_deg=False)
    out = _tc_layer(h, parts1, recipb, w1a_t, w1b_t, relu=False)
    return out


# speedup vs baseline: 1.1323x; 1.1323x over previous
import dataclasses
import functools

import jax
import jax.numpy as jnp
from jax import lax
from jax.experimental import pallas as pl
from jax.experimental.pallas import tpu as pltpu
from jax.experimental.pallas import tpu_sc as plsc

N = 10000
E = 320000
D = 128

NC = 2
NS = 16
NW = NC * NS
EPW = E // NW
W = 80
NWIN = EPW // W
K = 25
NCH = NWIN // K
L = 16


def _sc_agg(feat, src, dst, with_deg):
    nbuf = 3 if with_deg else 4
    mesh = plsc.VectorSubcoreMesh(core_axis_name="c", subcore_axis_name="s")
    out_type = [jax.ShapeDtypeStruct((NC, N, D), jnp.float32)]
    scratch = [
        pltpu.VMEM_SHARED((N, D), jnp.float32),
        pltpu.VMEM((K * W,), jnp.int32),
        pltpu.VMEM((K * W,), jnp.int32),
        pltpu.VMEM((W,), jnp.int32),
        pltpu.VMEM((W,), jnp.int32),
    ] + [pltpu.VMEM((W, D), jnp.float32) for _ in range(nbuf)] + [
        pltpu.SemaphoreType.DMA,
        pltpu.SemaphoreType.DMA,
    ]
    if with_deg:
        out_type.append(jax.ShapeDtypeStruct((NW, 1, N), jnp.float32))
        scratch.append(pltpu.VMEM((N,), jnp.float32))

    def body(*refs):
        (feat_h, src_h, dst_h, parts_h, *rest) = refs
        if with_deg:
            degp_h = rest[0]
            rest = rest[1:]
        (acc_sh, src_v, dst_v, dstw0, dstw1, *rest) = rest
        rows = rest[:nbuf]
        sem = rest[nbuf]
        sem_s = rest[nbuf + 1]
        if with_deg:
            hist_v = rest[nbuf + 2]
        dstw = [dstw0, dstw1]
        c = lax.axis_index("c")
        s = lax.axis_index("s")
        wid = s * NC + c

        @pl.loop(0, W)
        def _(i):
            @pl.loop(0, D, step=L)
            def _(j):
                rows[0].at[i, pl.ds(j, L)][...] = jnp.zeros((L,), jnp.float32)

        if with_deg:
            @pl.loop(0, N, step=L)
            def _(i):
                hist_v.at[pl.ds(i, L)][...] = jnp.zeros((L,), jnp.float32)

        nfull = (N // W) // NS

        def for_my_chunks(fn):
            for k in range(nfull):
                fn((s + k * NS) * W)
            tail = s + nfull * NS

            @pl.when(tail < N // W)
            def _():
                fn(tail * W)

        for_my_chunks(lambda lo: pltpu.async_copy(
            rows[0], acc_sh.at[pl.ds(lo, W)], sem))
        for_my_chunks(lambda lo: pltpu.make_async_copy(
            rows[0], acc_sh.at[pl.ds(lo, W)], sem).wait())

        plsc.subcore_barrier()

        ones = jnp.full((L,), 1.0, jnp.float32)

        def do_scatter(j, buf):
            dw = dstw[j % 2]
            for k in range(W // L):
                v = dst_v[pl.ds(j * W + k * L, L)]
                dw.at[pl.ds(k * L, L)][...] = v
                if with_deg:
                    plsc.addupdate_scatter(hist_v, [v], ones)
            return pltpu.async_copy(buf, acc_sh.at[dw], sem_s, add=True)

        def fire(j):
            return pltpu.async_copy(
                feat_h.at[src_v.at[pl.ds(j * W, W)]], rows[j % nbuf], sem)

        lead = 2
        depth = nbuf - lead

        @pl.loop(0, NCH)
        def _(g):
            base = wid * EPW + g * K * W
            pltpu.sync_copy(src_h.at[pl.ds(base, K * W)], src_v)
            pltpu.sync_copy(dst_h.at[pl.ds(base, K * W)], dst_v)
            gd = [fire(j) for j in range(lead)] + [None] * (nbuf - lead)
            sd = [None, None]
            for j in range(K):
                if j >= depth and sd[(j - depth) % 2] is not None:
                    sd[(j - depth) % 2].wait()
                    sd[(j - depth) % 2] = None
                if j + lead < K:
                    gd[(j + lead) % nbuf] = fire(j + lead)
                gd[j % nbuf].wait()
                sd[j % 2] = do_scatter(j, rows[j % nbuf])
            for dsc in sd:
                if dsc is not None:
                    dsc.wait()

        plsc.subcore_barrier()

        for_my_chunks(lambda lo: pltpu.async_copy(
            acc_sh.at[pl.ds(lo, W)], parts_h.at[c].at[pl.ds(lo, W)], sem))
        if with_deg:
            outd = pltpu.async_copy(hist_v, degp_h.at[wid].at[0], sem)
        for_my_chunks(lambda lo: pltpu.make_async_copy(
            acc_sh.at[pl.ds(lo, W)], parts_h.at[c].at[pl.ds(lo, W)],
            sem).wait())
        if with_deg:
            outd.wait()

    cp = pltpu.CompilerParams()
    if "needs_layout_passes" in pltpu.CompilerParams.__dataclass_fields__:
        cp = dataclasses.replace(cp, needs_layout_passes=False)
    fn = pl.kernel(body, out_type=tuple(out_type), mesh=mesh,
                   scratch_types=scratch, compiler_params=cp)
    return fn(feat, src, dst)


_BN = 1000


def _tc0_body(x_ref, p_ref, d_ref, wa_ref, wb_ref, o_ref, r_out_ref, recip_s):
    i = pl.program_id(0)

    @pl.when(i == 0)
    def _():
        d = d_ref[...][:, 0, :]
        ones = jnp.ones((NW, 1), jnp.float32)
        deg = lax.dot_general(d, ones, (((0,), (0,)), ((), ())),
                              preferred_element_type=jnp.float32)
        recip_s[...] = jnp.broadcast_to(1.0 / jnp.maximum(deg, 1.0), (N, D))

    r = recip_s[pl.ds(i * _BN, _BN), :]
    r_out_ref[...] = r
    agg = (p_ref[0] + p_ref[1]) * r
    y = (jnp.dot(x_ref[...], wa_ref[...], preferred_element_type=jnp.float32)
         + jnp.dot(agg, wb_ref[...], preferred_element_type=jnp.float32))
    o_ref[...] = jnp.maximum(y, 0.0)


def _tc_layer0(x, parts, degp, wa_t, wb_t):
    return pl.pallas_call(
        _tc0_body,
        grid=(N // _BN,),
        in_specs=[
            pl.BlockSpec((_BN, D), lambda i: (i, 0)),
            pl.BlockSpec((NC, _BN, D), lambda i: (0, i, 0)),
            pl.BlockSpec((NW, 1, N), lambda i: (0, 0, 0)),
            pl.BlockSpec((D, D), lambda i: (0, 0)),
            pl.BlockSpec((D, D), lambda i: (0, 0)),
        ],
        out_specs=[
            pl.BlockSpec((_BN, D), lambda i: (i, 0)),
            pl.BlockSpec((_BN, D), lambda i: (i, 0)),
        ],
        out_shape=[
            jax.ShapeDtypeStruct((N, D), jnp.float32),
            jax.ShapeDtypeStruct((N, D), jnp.float32),
        ],
        scratch_shapes=[pltpu.VMEM((N, D), jnp.float32)],
    )(x, parts, degp, wa_t, wb_t)


def _tc_body(relu, x_ref, p_ref, r_ref, wa_ref, wb_ref, o_ref):
    agg = (p_ref[0] + p_ref[1]) * r_ref[...]
    y = (jnp.dot(x_ref[...], wa_ref[...], preferred_element_type=jnp.float32)
         + jnp.dot(agg, wb_ref[...], preferred_element_type=jnp.float32))
    if relu:
        y = jnp.maximum(y, 0.0)
    o_ref[...] = y


def _tc_layer(x, parts, recipb, wa_t, wb_t, relu):
    grid = (N // _BN,)
    return pl.pallas_call(
        functools.partial(_tc_body, relu),
        grid=grid,
        in_specs=[
            pl.BlockSpec((_BN, D), lambda i: (i, 0)),
            pl.BlockSpec((NC, _BN, D), lambda i: (0, i, 0)),
            pl.BlockSpec((_BN, D), lambda i: (i, 0)),
            pl.BlockSpec((D, D), lambda i: (0, 0)),
            pl.BlockSpec((D, D), lambda i: (0, 0)),
        ],
        out_specs=pl.BlockSpec((_BN, D), lambda i: (i, 0)),
        out_shape=jax.ShapeDtypeStruct((N, D), jnp.float32),
    )(x, parts, recipb, wa_t, wb_t)


def kernel(feat, edge_index, W0, W1):
    src = edge_index[0]
    dst = edge_index[1]
    w0a_t = W0[:, :D].T
    w0b_t = W0[:, D:].T
    w1a_t = W1[:, :D].T
    w1b_t = W1[:, D:].T

    parts0, degp = _sc_agg(feat, src, dst, with_deg=True)
    h, recipb = _tc_layer0(feat, parts0, degp, w0a_t, w0b_t)
    parts1, = _sc_agg(h, src, dst, with_deg=False)
    out = _tc_layer(h, parts1, recipb, w1a_t, w1b_t, relu=False)
    return out


# speedup vs baseline: 1.1765x; 1.0390x over previous
import dataclasses
import functools

import jax
import jax.numpy as jnp
from jax import lax
from jax.experimental import pallas as pl
from jax.experimental.pallas import tpu as pltpu
from jax.experimental.pallas import tpu_sc as plsc

N = 10000
E = 320000
D = 128

NC = 2
NS = 16
NW = NC * NS
EPW = E // NW
W = 80
NWIN = EPW // W
K = 25
NCH = NWIN // K
L = 16


def _sc_agg(feat, src, dst, with_deg):
    nbuf = 3 if with_deg else 4
    mesh = plsc.VectorSubcoreMesh(core_axis_name="c", subcore_axis_name="s")
    out_type = [jax.ShapeDtypeStruct((NC, N, D), jnp.float32)]
    scratch = [
        pltpu.VMEM_SHARED((N, D), jnp.float32),
        pltpu.VMEM((2 * K * W,), jnp.int32),
        pltpu.VMEM((2 * K * W,), jnp.int32),
        pltpu.VMEM((W,), jnp.int32),
        pltpu.VMEM((W,), jnp.int32),
    ] + [pltpu.VMEM((W, D), jnp.float32) for _ in range(nbuf)] + [
        pltpu.SemaphoreType.DMA,
        pltpu.SemaphoreType.DMA,
        pltpu.SemaphoreType.DMA,
    ]
    if with_deg:
        out_type.append(jax.ShapeDtypeStruct((NW, 1, N), jnp.float32))
        scratch.append(pltpu.VMEM((N,), jnp.float32))

    def body(*refs):
        (feat_h, src_h, dst_h, parts_h, *rest) = refs
        if with_deg:
            degp_h = rest[0]
            rest = rest[1:]
        (acc_sh, src_v, dst_v, dstw0, dstw1, *rest) = rest
        rows = rest[:nbuf]
        sem = rest[nbuf]
        sem_s = rest[nbuf + 1]
        sem_i = rest[nbuf + 2]
        if with_deg:
            hist_v = rest[nbuf + 3]
        dstw = [dstw0, dstw1]
        c = lax.axis_index("c")
        s = lax.axis_index("s")
        wid = s * NC + c

        @pl.loop(0, W)
        def _(i):
            @pl.loop(0, D, step=L)
            def _(j):
                rows[0].at[i, pl.ds(j, L)][...] = jnp.zeros((L,), jnp.float32)

        if with_deg:
            @pl.loop(0, N, step=L)
            def _(i):
                hist_v.at[pl.ds(i, L)][...] = jnp.zeros((L,), jnp.float32)

        nfull = (N // W) // NS

        def for_my_chunks(fn):
            for k in range(nfull):
                fn((s + k * NS) * W)
            tail = s + nfull * NS

            @pl.when(tail < N // W)
            def _():
                fn(tail * W)

        for_my_chunks(lambda lo: pltpu.async_copy(
            rows[0], acc_sh.at[pl.ds(lo, W)], sem))
        for_my_chunks(lambda lo: pltpu.make_async_copy(
            rows[0], acc_sh.at[pl.ds(lo, W)], sem).wait())

        plsc.subcore_barrier()

        ones = jnp.full((L,), 1.0, jnp.float32)

        def do_scatter(j, buf, ioff):
            dw = dstw[j % 2]
            for k in range(W // L):
                v = dst_v[pl.ds(ioff + j * W + k * L, L)]
                dw.at[pl.ds(k * L, L)][...] = v
                if with_deg:
                    plsc.addupdate_scatter(hist_v, [v], ones)
            return pltpu.async_copy(buf, acc_sh.at[dw], sem_s, add=True)

        def fire(j, ioff):
            return pltpu.async_copy(
                feat_h.at[src_v.at[pl.ds(ioff + j * W, W)]], rows[j % nbuf],
                sem)

        def idx_copies(g, off):
            base = wid * EPW + g * K * W
            return [
                pltpu.make_async_copy(src_h.at[pl.ds(base, K * W)],
                                      src_v.at[pl.ds(off, K * W)], sem_i),
                pltpu.make_async_copy(dst_h.at[pl.ds(base, K * W)],
                                      dst_v.at[pl.ds(off, K * W)], sem_i),
            ]

        lead = 2
        depth = nbuf - lead

        for cp_ in idx_copies(0, 0):
            cp_.start()

        @pl.loop(0, NCH)
        def _(g):
            ioff = (g % 2) * (K * W)
            for cp_ in idx_copies(g, ioff):
                cp_.wait()

            @pl.when(g + 1 < NCH)
            def _():
                for cp_ in idx_copies(g + 1, (K * W) - ioff):
                    cp_.start()

            gd = ([fire(j, ioff) for j in range(lead)]
                  + [None] * (nbuf - lead))
            sd = [None, None]
            for j in range(K):
                if j >= depth and sd[(j - depth) % 2] is not None:
                    sd[(j - depth) % 2].wait()
                    sd[(j - depth) % 2] = None
                if j + lead < K:
                    gd[(j + lead) % nbuf] = fire(j + lead, ioff)
                gd[j % nbuf].wait()
                sd[j % 2] = do_scatter(j, rows[j % nbuf], ioff)
            for dsc in sd:
                if dsc is not None:
                    dsc.wait()

        plsc.subcore_barrier()

        for_my_chunks(lambda lo: pltpu.async_copy(
            acc_sh.at[pl.ds(lo, W)], parts_h.at[c].at[pl.ds(lo, W)], sem))
        if with_deg:
            outd = pltpu.async_copy(hist_v, degp_h.at[wid].at[0], sem)
        for_my_chunks(lambda lo: pltpu.make_async_copy(
            acc_sh.at[pl.ds(lo, W)], parts_h.at[c].at[pl.ds(lo, W)],
            sem).wait())
        if with_deg:
            outd.wait()

    cp = pltpu.CompilerParams()
    if "needs_layout_passes" in pltpu.CompilerParams.__dataclass_fields__:
        cp = dataclasses.replace(cp, needs_layout_passes=False)
    fn = pl.kernel(body, out_type=tuple(out_type), mesh=mesh,
                   scratch_types=scratch, compiler_params=cp)
    return fn(feat, src, dst)


_BN = 1000


def _tc0_body(x_ref, p_ref, d_ref, wa_ref, wb_ref, o_ref, r_out_ref, recip_s):
    i = pl.program_id(0)

    @pl.when(i == 0)
    def _():
        d = d_ref[...][:, 0, :]
        ones = jnp.ones((NW, 1), jnp.float32)
        deg = lax.dot_general(d, ones, (((0,), (0,)), ((), ())),
                              preferred_element_type=jnp.float32)
        recip_s[...] = jnp.broadcast_to(1.0 / jnp.maximum(deg, 1.0), (N, D))

    r = recip_s[pl.ds(i * _BN, _BN), :]
    r_out_ref[...] = r
    agg = (p_ref[0] + p_ref[1]) * r
    y = (jnp.dot(x_ref[...], wa_ref[...], preferred_element_type=jnp.float32)
         + jnp.dot(agg, wb_ref[...], preferred_element_type=jnp.float32))
    o_ref[...] = jnp.maximum(y, 0.0)


def _tc_layer0(x, parts, degp, wa_t, wb_t):
    return pl.pallas_call(
        _tc0_body,
        grid=(N // _BN,),
        in_specs=[
            pl.BlockSpec((_BN, D), lambda i: (i, 0)),
            pl.BlockSpec((NC, _BN, D), lambda i: (0, i, 0)),
            pl.BlockSpec((NW, 1, N), lambda i: (0, 0, 0)),
            pl.BlockSpec((D, D), lambda i: (0, 0)),
            pl.BlockSpec((D, D), lambda i: (0, 0)),
        ],
        out_specs=[
            pl.BlockSpec((_BN, D), lambda i: (i, 0)),
            pl.BlockSpec((_BN, D), lambda i: (i, 0)),
        ],
        out_shape=[
            jax.ShapeDtypeStruct((N, D), jnp.float32),
            jax.ShapeDtypeStruct((N, D), jnp.float32),
        ],
        scratch_shapes=[pltpu.VMEM((N, D), jnp.float32)],
    )(x, parts, degp, wa_t, wb_t)


def _tc_body(relu, x_ref, p_ref, r_ref, wa_ref, wb_ref, o_ref):
    agg = (p_ref[0] + p_ref[1]) * r_ref[...]
    y = (jnp.dot(x_ref[...], wa_ref[...], preferred_element_type=jnp.float32)
         + jnp.dot(agg, wb_ref[...], preferred_element_type=jnp.float32))
    if relu:
        y = jnp.maximum(y, 0.0)
    o_ref[...] = y


def _tc_layer(x, parts, recipb, wa_t, wb_t, relu):
    grid = (N // _BN,)
    return pl.pallas_call(
        functools.partial(_tc_body, relu),
        grid=grid,
        in_specs=[
            pl.BlockSpec((_BN, D), lambda i: (i, 0)),
            pl.BlockSpec((NC, _BN, D), lambda i: (0, i, 0)),
            pl.BlockSpec((_BN, D), lambda i: (i, 0)),
            pl.BlockSpec((D, D), lambda i: (0, 0)),
            pl.BlockSpec((D, D), lambda i: (0, 0)),
        ],
        out_specs=pl.BlockSpec((_BN, D), lambda i: (i, 0)),
        out_shape=jax.ShapeDtypeStruct((N, D), jnp.float32),
    )(x, parts, recipb, wa_t, wb_t)


def kernel(feat, edge_index, W0, W1):
    src = edge_index[0]
    dst = edge_index[1]
    w0a_t = W0[:, :D].T
    w0b_t = W0[:, D:].T
    w1a_t = W1[:, :D].T
    w1b_t = W1[:, D:].T

    parts0, degp = _sc_agg(feat, src, dst, with_deg=True)
    h, recipb = _tc_layer0(feat, parts0, degp, w0a_t, w0b_t)
    parts1, = _sc_agg(h, src, dst, with_deg=False)
    out = _tc_layer(h, parts1, recipb, w1a_t, w1b_t, relu=False)
    return out


# speedup vs baseline: 1.1774x; 1.0008x over previous
import dataclasses
import functools

import jax
import jax.numpy as jnp
from jax import lax
from jax.experimental import pallas as pl
from jax.experimental.pallas import tpu as pltpu
from jax.experimental.pallas import tpu_sc as plsc

N = 10000
E = 320000
D = 128

NC = 2
NS = 16
NW = NC * NS
EPW = E // NW
W = 80
NWIN = EPW // W
K = 25
NCH = NWIN // K
L = 16


def _sc_agg(feat, src, dst, with_deg):
    nbuf = 3 if with_deg else 4
    mesh = plsc.VectorSubcoreMesh(core_axis_name="c", subcore_axis_name="s")
    out_type = [jax.ShapeDtypeStruct((NC, N, D), jnp.float32)]
    scratch = [
        pltpu.VMEM_SHARED((N, D), jnp.float32),
        pltpu.VMEM((2 * K * W,), jnp.int32),
        pltpu.VMEM((2 * K * W,), jnp.int32),
        pltpu.VMEM((W,), jnp.int32),
        pltpu.VMEM((W,), jnp.int32),
    ] + [pltpu.VMEM((W, D), jnp.float32) for _ in range(nbuf)] + [
        pltpu.SemaphoreType.DMA,
        pltpu.SemaphoreType.DMA,
        pltpu.SemaphoreType.DMA,
    ]
    if with_deg:
        out_type.append(jax.ShapeDtypeStruct((NW, 1, N), jnp.float32))
        scratch.append(pltpu.VMEM((N,), jnp.float32))

    def body(*refs):
        (feat_h, src_h, dst_h, parts_h, *rest) = refs
        if with_deg:
            degp_h = rest[0]
            rest = rest[1:]
        (acc_sh, src_v, dst_v, dstw0, dstw1, *rest) = rest
        rows = rest[:nbuf]
        sem = rest[nbuf]
        sem_s = rest[nbuf + 1]
        sem_i = rest[nbuf + 2]
        if with_deg:
            hist_v = rest[nbuf + 3]
        dstw = [dstw0, dstw1]
        c = lax.axis_index("c")
        s = lax.axis_index("s")
        wid = s * NC + c

        def idx_copies_early():
            base0 = wid * EPW
            pltpu.async_copy(src_h.at[pl.ds(base0, K * W)],
                             src_v.at[pl.ds(0, K * W)], sem_i)
            pltpu.async_copy(dst_h.at[pl.ds(base0, K * W)],
                             dst_v.at[pl.ds(0, K * W)], sem_i)

        idx_copies_early()

        @pl.loop(0, W)
        def _(i):
            @pl.loop(0, D, step=L)
            def _(j):
                rows[0].at[i, pl.ds(j, L)][...] = jnp.zeros((L,), jnp.float32)

        if with_deg:
            @pl.loop(0, N, step=L)
            def _(i):
                hist_v.at[pl.ds(i, L)][...] = jnp.zeros((L,), jnp.float32)

        nfull = (N // W) // NS

        def for_my_chunks(fn):
            for k in range(nfull):
                fn((s + k * NS) * W)
            tail = s + nfull * NS

            @pl.when(tail < N // W)
            def _():
                fn(tail * W)

        for_my_chunks(lambda lo: pltpu.async_copy(
            rows[0], acc_sh.at[pl.ds(lo, W)], sem))
        for_my_chunks(lambda lo: pltpu.make_async_copy(
            rows[0], acc_sh.at[pl.ds(lo, W)], sem).wait())

        plsc.subcore_barrier()

        ones = jnp.full((L,), 1.0, jnp.float32)

        def do_scatter(j, buf, ioff):
            dw = dstw[j % 2]
            for k in range(W // L):
                v = dst_v[pl.ds(ioff + j * W + k * L, L)]
                dw.at[pl.ds(k * L, L)][...] = v
                if with_deg:
                    plsc.addupdate_scatter(hist_v, [v], ones)
            return pltpu.async_copy(buf, acc_sh.at[dw], sem_s, add=True)

        def fire(j, ioff):
            return pltpu.async_copy(
                feat_h.at[src_v.at[pl.ds(ioff + j * W, W)]], rows[j % nbuf],
                sem)

        def idx_copies(g, off):
            base = wid * EPW + g * K * W
            return [
                pltpu.make_async_copy(src_h.at[pl.ds(base, K * W)],
                                      src_v.at[pl.ds(off, K * W)], sem_i),
                pltpu.make_async_copy(dst_h.at[pl.ds(base, K * W)],
                                      dst_v.at[pl.ds(off, K * W)], sem_i),
            ]

        lead = 2 if with_deg else 3
        depth = nbuf - lead

        @pl.loop(0, NCH)
        def _(g):
            ioff = (g % 2) * (K * W)
            for cp_ in idx_copies(g, ioff):
                cp_.wait()

            @pl.when(g + 1 < NCH)
            def _():
                for cp_ in idx_copies(g + 1, (K * W) - ioff):
                    cp_.start()

            gd = ([fire(j, ioff) for j in range(lead)]
                  + [None] * (nbuf - lead))
            sd = [None, None]
            for j in range(K):
                if j >= depth and sd[(j - depth) % 2] is not None:
                    sd[(j - depth) % 2].wait()
                    sd[(j - depth) % 2] = None
                if j + lead < K:
                    gd[(j + lead) % nbuf] = fire(j + lead, ioff)
                gd[j % nbuf].wait()
                sd[j % 2] = do_scatter(j, rows[j % nbuf], ioff)
            for dsc in sd:
                if dsc is not None:
                    dsc.wait()

        plsc.subcore_barrier()

        for_my_chunks(lambda lo: pltpu.async_copy(
            acc_sh.at[pl.ds(lo, W)], parts_h.at[c].at[pl.ds(lo, W)], sem))
        if with_deg:
            outd = pltpu.async_copy(hist_v, degp_h.at[wid].at[0], sem)
        for_my_chunks(lambda lo: pltpu.make_async_copy(
            acc_sh.at[pl.ds(lo, W)], parts_h.at[c].at[pl.ds(lo, W)],
            sem).wait())
        if with_deg:
            outd.wait()

    cp = pltpu.CompilerParams()
    if "needs_layout_passes" in pltpu.CompilerParams.__dataclass_fields__:
        cp = dataclasses.replace(cp, needs_layout_passes=False)
    fn = pl.kernel(body, out_type=tuple(out_type), mesh=mesh,
                   scratch_types=scratch, compiler_params=cp)
    return fn(feat, src, dst)


_BN = 1000


def _tc0_body(x_ref, p_ref, d_ref, wa_ref, wb_ref, o_ref, r_out_ref, recip_s):
    i = pl.program_id(0)

    @pl.when(i == 0)
    def _():
        d = d_ref[...][:, 0, :]
        ones = jnp.ones((NW, 1), jnp.float32)
        deg = lax.dot_general(d, ones, (((0,), (0,)), ((), ())),
                              preferred_element_type=jnp.float32)
        recip_s[...] = jnp.broadcast_to(1.0 / jnp.maximum(deg, 1.0), (N, D))

    r = recip_s[pl.ds(i * _BN, _BN), :]
    r_out_ref[...] = r
    agg = (p_ref[0] + p_ref[1]) * r
    y = (jnp.dot(x_ref[...], wa_ref[...], preferred_element_type=jnp.float32)
         + jnp.dot(agg, wb_ref[...], preferred_element_type=jnp.float32))
    o_ref[...] = jnp.maximum(y, 0.0)


def _tc_layer0(x, parts, degp, wa_t, wb_t):
    return pl.pallas_call(
        _tc0_body,
        grid=(N // _BN,),
        in_specs=[
            pl.BlockSpec((_BN, D), lambda i: (i, 0)),
            pl.BlockSpec((NC, _BN, D), lambda i: (0, i, 0)),
            pl.BlockSpec((NW, 1, N), lambda i: (0, 0, 0)),
            pl.BlockSpec((D, D), lambda i: (0, 0)),
            pl.BlockSpec((D, D), lambda i: (0, 0)),
        ],
        out_specs=[
            pl.BlockSpec((_BN, D), lambda i: (i, 0)),
            pl.BlockSpec((_BN, D), lambda i: (i, 0)),
        ],
        out_shape=[
            jax.ShapeDtypeStruct((N, D), jnp.float32),
            jax.ShapeDtypeStruct((N, D), jnp.float32),
        ],
        scratch_shapes=[pltpu.VMEM((N, D), jnp.float32)],
    )(x, parts, degp, wa_t, wb_t)


def _tc_body(relu, x_ref, p_ref, r_ref, wa_ref, wb_ref, o_ref):
    agg = (p_ref[0] + p_ref[1]) * r_ref[...]
    y = (jnp.dot(x_ref[...], wa_ref[...], preferred_element_type=jnp.float32)
         + jnp.dot(agg, wb_ref[...], preferred_element_type=jnp.float32))
    if relu:
        y = jnp.maximum(y, 0.0)
    o_ref[...] = y


def _tc_layer(x, parts, recipb, wa_t, wb_t, relu):
    grid = (N // _BN,)
    return pl.pallas_call(
        functools.partial(_tc_body, relu),
        grid=grid,
        in_specs=[
            pl.BlockSpec((_BN, D), lambda i: (i, 0)),
            pl.BlockSpec((NC, _BN, D), lambda i: (0, i, 0)),
            pl.BlockSpec((_BN, D), lambda i: (i, 0)),
            pl.BlockSpec((D, D), lambda i: (0, 0)),
            pl.BlockSpec((D, D), lambda i: (0, 0)),
        ],
        out_specs=pl.BlockSpec((_BN, D), lambda i: (i, 0)),
        out_shape=jax.ShapeDtypeStruct((N, D), jnp.float32),
    )(x, parts, recipb, wa_t, wb_t)


def kernel(feat, edge_index, W0, W1):
    src = edge_index[0]
    dst = edge_index[1]
    w0a_t = W0[:, :D].T
    w0b_t = W0[:, D:].T
    w1a_t = W1[:, :D].T
    w1b_t = W1[:, D:].T

    parts0, degp = _sc_agg(feat, src, dst, with_deg=True)
    h, recipb = _tc_layer0(feat, parts0, degp, w0a_t, w0b_t)
    parts1, = _sc_agg(h, src, dst, with_deg=False)
    out = _tc_layer(h, parts1, recipb, w1a_t, w1b_t, relu=False)
    return out
